# baseline (speedup 1.0000x reference)
import jax
import jax.numpy as jnp
from jax.experimental import pallas as pl
from jax.experimental.pallas import tpu as pltpu

B = 16
T = 512
I = 512
H = 512
T_CHUNK = 128


def _rnn_chunk_kernel(x_ref, wih_ref, whh_ref, bias_ref, out_ref, h_ref, xw_ref):
    @pl.when(pl.program_id(0) == 0)
    def _():
        h_ref[...] = jnp.zeros_like(h_ref)

    x = x_ref[...].reshape(B * T_CHUNK, I)
    xw = jax.lax.dot_general(
        x, wih_ref[...], (((1,), (1,)), ((), ())),
        preferred_element_type=jnp.float32,
    )
    xw_ref[...] = xw.reshape(B, T_CHUNK, H) + bias_ref[...].reshape(1, 1, H)

    whh = whh_ref[...]
    dn = (((1,), (1,)), ((), ()))

    def body(i, carry):
        ha, hb = carry
        acca = jax.lax.dot_general(ha, whh, dn, preferred_element_type=jnp.float32)
        accb = jax.lax.dot_general(hb, whh, dn, preferred_element_type=jnp.float32)
        han = jnp.tanh(acca + xw_ref[0:8, i, :])
        hbn = jnp.tanh(accb + xw_ref[8:16, i, :])
        out_ref[0:8, pl.ds(i * H, H)] = han
        out_ref[8:16, pl.ds(i * H, H)] = hbn
        return (han, hbn)

    h0 = h_ref[...]
    haf, hbf = jax.lax.fori_loop(0, T_CHUNK, body, (h0[0:8], h0[8:16]), unroll=16)
    h_ref[0:8, :] = haf
    h_ref[8:16, :] = hbf


def kernel(batch, W_ih, W_hh, b):
    bias2d = b.reshape(1, H)

    out2d = pl.pallas_call(
        _rnn_chunk_kernel,
        grid=(T // T_CHUNK,),
        in_specs=[
            pl.BlockSpec((B, T_CHUNK, I), lambda c: (0, c, 0)),
            pl.BlockSpec((H, I), lambda c: (0, 0)),
            pl.BlockSpec((H, H), lambda c: (0, 0)),
            pl.BlockSpec((1, H), lambda c: (0, 0)),
        ],
        out_specs=pl.BlockSpec((B, T_CHUNK * H), lambda c: (0, c)),
        out_shape=jax.ShapeDtypeStruct((B, T * H), jnp.float32),
        scratch_shapes=[
            pltpu.VMEM((B, H), jnp.float32),
            pltpu.VMEM((B, T_CHUNK, H), jnp.float32),
        ],
    )(batch, W_ih, W_hh, bias2d)

    outputs = out2d.reshape(B, T, H)
    hT = outputs[:, -1, :]
    return outputs, hT


# speedup vs baseline: 1.2908x; 1.2908x over previous
import jax
import jax.numpy as jnp
from jax.experimental import pallas as pl
from jax.experimental.pallas import tpu as pltpu

B = 16
T = 512
I = 512
H = 512
T_CHUNK = 128


def _rnn_chunk_kernel(x_ref, wih_ref, whh_ref, bias_ref, out_ref, h_ref, xw_ref):
    @pl.when(pl.program_id(0) == 0)
    def _():
        h_ref[...] = jnp.zeros_like(h_ref)

    x = x_ref[...].reshape(B * T_CHUNK, I)
    xw = jax.lax.dot_general(
        x, wih_ref[...], (((1,), (1,)), ((), ())),
        preferred_element_type=jnp.float32,
    )
    xw_ref[...] = xw.reshape(B, T_CHUNK, H) + bias_ref[...].reshape(1, 1, H)

    whh = whh_ref[...].astype(jnp.bfloat16)

    def body(i, h):
        acc = jax.lax.dot_general(
            h.astype(jnp.bfloat16), whh, (((1,), (1,)), ((), ())),
            preferred_element_type=jnp.float32,
        )
        h_new = jnp.tanh(acc + xw_ref[:, i, :])
        out_ref[:, pl.ds(i * H, H)] = h_new
        return h_new

    h_ref[...] = jax.lax.fori_loop(0, T_CHUNK, body, h_ref[...], unroll=16)


def kernel(batch, W_ih, W_hh, b):
    bias2d = b.reshape(1, H)

    out2d = pl.pallas_call(
        _rnn_chunk_kernel,
        grid=(T // T_CHUNK,),
        in_specs=[
            pl.BlockSpec((B, T_CHUNK, I), lambda c: (0, c, 0)),
            pl.BlockSpec((H, I), lambda c: (0, 0)),
            pl.BlockSpec((H, H), lambda c: (0, 0)),
            pl.BlockSpec((1, H), lambda c: (0, 0)),
        ],
        out_specs=pl.BlockSpec((B, T_CHUNK * H), lambda c: (0, c)),
        out_shape=jax.ShapeDtypeStruct((B, T * H), jnp.float32),
        scratch_shapes=[
            pltpu.VMEM((B, H), jnp.float32),
            pltpu.VMEM((B, T_CHUNK, H), jnp.float32),
        ],
    )(batch, W_ih, W_hh, bias2d)

    outputs = out2d.reshape(B, T, H)
    hT = outputs[:, -1, :]
    return outputs, hT


# speedup vs baseline: 1.3165x; 1.0200x over previous
import jax
import jax.numpy as jnp
from jax.experimental import pallas as pl
from jax.experimental.pallas import tpu as pltpu

B = 16
T = 512
I = 512
H = 512
T_CHUNK = 128


def _rnn_chunk_kernel(x_ref, wih_ref, whh_ref, bias_ref, out_ref, h_ref, xw_ref):
    @pl.when(pl.program_id(0) == 0)
    def _():
        h_ref[...] = jnp.zeros_like(h_ref)

    x = x_ref[...].reshape(B * T_CHUNK, I)
    xw = jax.lax.dot_general(
        x, wih_ref[...], (((1,), (1,)), ((), ())),
        preferred_element_type=jnp.float32,
    )
    xw_ref[...] = xw.reshape(B, T_CHUNK, H) + bias_ref[...].reshape(1, 1, H)

    whh = whh_ref[...]

    def body(i, h):
        acc = jax.lax.dot_general(
            h, whh, (((1,), (1,)), ((), ())),
            preferred_element_type=jnp.float32,
        )
        h_new = jnp.tanh(acc + xw_ref[:, i, :])
        out_ref[:, pl.ds(i * H, H)] = h_new
        return h_new

    h_ref[...] = jax.lax.fori_loop(0, T_CHUNK, body, h_ref[...], unroll=32)


def kernel(batch, W_ih, W_hh, b):
    bias2d = b.reshape(1, H)

    out2d = pl.pallas_call(
        _rnn_chunk_kernel,
        grid=(T // T_CHUNK,),
        in_specs=[
            pl.BlockSpec((B, T_CHUNK, I), lambda c: (0, c, 0)),
            pl.BlockSpec((H, I), lambda c: (0, 0)),
            pl.BlockSpec((H, H), lambda c: (0, 0)),
            pl.BlockSpec((1, H), lambda c: (0, 0)),
        ],
        out_specs=pl.BlockSpec((B, T_CHUNK * H), lambda c: (0, c)),
        out_shape=jax.ShapeDtypeStruct((B, T * H), jnp.float32),
        scratch_shapes=[
            pltpu.VMEM((B, H), jnp.float32),
            pltpu.VMEM((B, T_CHUNK, H), jnp.float32),
        ],
    )(batch, W_ih, W_hh, bias2d)

    outputs = out2d.reshape(B, T, H)
    hT = outputs[:, -1, :]
    return outputs, hT


# speedup vs baseline: 1.3254x; 1.0067x over previous
import jax
import jax.numpy as jnp
from jax.experimental import pallas as pl
from jax.experimental.pallas import tpu as pltpu

B = 16
T = 512
I = 512
H = 512
T_CHUNK = 128


def _rnn_chunk_kernel(x_ref, wih_ref, whh_ref, bias_ref, out_ref, h_ref, xw_ref):
    @pl.when(pl.program_id(0) == 0)
    def _():
        h_ref[...] = jnp.zeros_like(h_ref)

    x = x_ref[...].reshape(B * T_CHUNK, I)
    xw = jax.lax.dot_general(
        x, wih_ref[...], (((1,), (1,)), ((), ())),
        preferred_element_type=jnp.float32,
    )
    xw_ref[...] = xw.reshape(B, T_CHUNK, H) + bias_ref[...].reshape(1, 1, H)

    whh = whh_ref[...]

    def body(i, h):
        acc = jax.lax.dot_general(
            h, whh, (((1,), (1,)), ((), ())),
            preferred_element_type=jnp.float32,
        )
        h_new = jnp.tanh(acc + xw_ref[:, i, :])
        out_ref[:, pl.ds(i * H, H)] = h_new
        return h_new

    h_ref[...] = jax.lax.fori_loop(0, T_CHUNK, body, h_ref[...], unroll=64)


def kernel(batch, W_ih, W_hh, b):
    bias2d = b.reshape(1, H)

    out2d = pl.pallas_call(
        _rnn_chunk_kernel,
        grid=(T // T_CHUNK,),
        in_specs=[
            pl.BlockSpec((B, T_CHUNK, I), lambda c: (0, c, 0)),
            pl.BlockSpec((H, I), lambda c: (0, 0)),
            pl.BlockSpec((H, H), lambda c: (0, 0)),
            pl.BlockSpec((1, H), lambda c: (0, 0)),
        ],
        out_specs=pl.BlockSpec((B, T_CHUNK * H), lambda c: (0, c)),
        out_shape=jax.ShapeDtypeStruct((B, T * H), jnp.float32),
        scratch_shapes=[
            pltpu.VMEM((B, H), jnp.float32),
            pltpu.VMEM((B, T_CHUNK, H), jnp.float32),
        ],
    )(batch, W_ih, W_hh, bias2d)

    outputs = out2d.reshape(B, T, H)
    hT = outputs[:, -1, :]
    return outputs, hT


# speedup vs baseline: 1.3348x; 1.0071x over previous
import jax
import jax.numpy as jnp
from jax.experimental import pallas as pl
from jax.experimental.pallas import tpu as pltpu

B = 16
T = 512
I = 512
H = 512
T_CHUNK = 128


def _rnn_chunk_kernel(x_ref, wih_ref, whh_ref, bias_ref, out_ref, h_ref, xw_ref):
    @pl.when(pl.program_id(0) == 0)
    def _():
        h_ref[...] = jnp.zeros_like(h_ref)

    x = x_ref[...].reshape(B * T_CHUNK, I)
    xw = jax.lax.dot_general(
        x, wih_ref[...], (((1,), (1,)), ((), ())),
        preferred_element_type=jnp.float32,
    )
    xw_ref[...] = xw.reshape(B, T_CHUNK, H) + bias_ref[...].reshape(1, 1, H)

    whh = whh_ref[...]

    def body(i, h):
        acc = jax.lax.dot_general(
            h, whh, (((1,), (1,)), ((), ())),
            preferred_element_type=jnp.float32,
        )
        h_new = jnp.tanh(acc + xw_ref[:, i, :])
        out_ref[:, pl.ds(i * H, H)] = h_new
        return h_new

    h_ref[...] = jax.lax.fori_loop(0, T_CHUNK, body, h_ref[...], unroll=128)


def kernel(batch, W_ih, W_hh, b):
    bias2d = b.reshape(1, H)

    out2d = pl.pallas_call(
        _rnn_chunk_kernel,
        grid=(T // T_CHUNK,),
        in_specs=[
            pl.BlockSpec((B, T_CHUNK, I), lambda c: (0, c, 0)),
            pl.BlockSpec((H, I), lambda c: (0, 0)),
            pl.BlockSpec((H, H), lambda c: (0, 0)),
            pl.BlockSpec((1, H), lambda c: (0, 0)),
        ],
        out_specs=pl.BlockSpec((B, T_CHUNK * H), lambda c: (0, c)),
        out_shape=jax.ShapeDtypeStruct((B, T * H), jnp.float32),
        scratch_shapes=[
            pltpu.VMEM((B, H), jnp.float32),
            pltpu.VMEM((B, T_CHUNK, H), jnp.float32),
        ],
    )(batch, W_ih, W_hh, bias2d)

    outputs = out2d.reshape(B, T, H)
    hT = outputs[:, -1, :]
    return outputs, hT


# speedup vs baseline: 1.3351x; 1.0003x over previous
import jax
import jax.numpy as jnp
from jax.experimental import pallas as pl
from jax.experimental.pallas import tpu as pltpu

B = 16
T = 512
I = 512
H = 512
T_CHUNK = 128


def _rnn_chunk_kernel(x_ref, wih_ref, whh_ref, bias_ref, out_ref, h_ref, xw_ref):
    @pl.when(pl.program_id(0) == 0)
    def _():
        h_ref[...] = jnp.zeros_like(h_ref)

    x = x_ref[...].reshape(B * T_CHUNK, I)
    xw = jax.lax.dot_general(
        x, wih_ref[...], (((1,), (1,)), ((), ())),
        preferred_element_type=jnp.float32,
    )
    xw_ref[...] = xw.reshape(B, T_CHUNK, H) + bias_ref[...].reshape(1, 1, H)

    def body(i, h):
        acc = jax.lax.dot_general(
            h, whh_ref[...], (((1,), (1,)), ((), ())),
            preferred_element_type=jnp.float32,
        )
        h_new = jnp.tanh(acc + xw_ref[:, i, :])
        out_ref[:, pl.ds(i * H, H)] = h_new
        return h_new

    h_ref[...] = jax.lax.fori_loop(0, T_CHUNK, body, h_ref[...], unroll=128)


def kernel(batch, W_ih, W_hh, b):
    bias2d = b.reshape(1, H)

    out2d = pl.pallas_call(
        _rnn_chunk_kernel,
        grid=(T // T_CHUNK,),
        in_specs=[
            pl.BlockSpec((B, T_CHUNK, I), lambda c: (0, c, 0)),
            pl.BlockSpec((H, I), lambda c: (0, 0)),
            pl.BlockSpec((H, H), lambda c: (0, 0)),
            pl.BlockSpec((1, H), lambda c: (0, 0)),
        ],
        out_specs=pl.BlockSpec((B, T_CHUNK * H), lambda c: (0, c)),
        out_shape=jax.ShapeDtypeStruct((B, T * H), jnp.float32),
        scratch_shapes=[
            pltpu.VMEM((B, H), jnp.float32),
            pltpu.VMEM((B, T_CHUNK, H), jnp.float32),
        ],
    )(batch, W_ih, W_hh, bias2d)

    outputs = out2d.reshape(B, T, H)
    hT = outputs[:, -1, :]
    return outputs, hT


# speedup vs baseline: 1.3417x; 1.0049x over previous
import jax
import jax.numpy as jnp
from jax.experimental import pallas as pl
from jax.experimental.pallas import tpu as pltpu

B = 16
T = 512
I = 512
H = 512
T_CHUNK = 64


def _rnn_chunk_kernel(x_ref, wih_ref, whh_ref, bias_ref, out_ref, h_ref, xw_ref):
    @pl.when(pl.program_id(0) == 0)
    def _():
        h_ref[...] = jnp.zeros_like(h_ref)

    x = x_ref[...].reshape(B * T_CHUNK, I)
    xw = jax.lax.dot_general(
        x, wih_ref[...], (((1,), (1,)), ((), ())),
        preferred_element_type=jnp.float32,
    )
    xw_ref[...] = xw.reshape(B, T_CHUNK, H) + bias_ref[...].reshape(1, 1, H)

    def body(i, h):
        acc = jax.lax.dot_general(
            h, whh_ref[...], (((1,), (1,)), ((), ())),
            preferred_element_type=jnp.float32,
        )
        h_new = jnp.tanh(acc + xw_ref[:, i, :])
        out_ref[:, pl.ds(i * H, H)] = h_new
        return h_new

    h_ref[...] = jax.lax.fori_loop(0, T_CHUNK, body, h_ref[...], unroll=64)


def kernel(batch, W_ih, W_hh, b):
    bias2d = b.reshape(1, H)

    out2d = pl.pallas_call(
        _rnn_chunk_kernel,
        grid=(T // T_CHUNK,),
        in_specs=[
            pl.BlockSpec((B, T_CHUNK, I), lambda c: (0, c, 0)),
            pl.BlockSpec((H, I), lambda c: (0, 0)),
            pl.BlockSpec((H, H), lambda c: (0, 0)),
            pl.BlockSpec((1, H), lambda c: (0, 0)),
        ],
        out_specs=pl.BlockSpec((B, T_CHUNK * H), lambda c: (0, c)),
        out_shape=jax.ShapeDtypeStruct((B, T * H), jnp.float32),
        scratch_shapes=[
            pltpu.VMEM((B, H), jnp.float32),
            pltpu.VMEM((B, T_CHUNK, H), jnp.float32),
        ],
    )(batch, W_ih, W_hh, bias2d)

    outputs = out2d.reshape(B, T, H)
    hT = outputs[:, -1, :]
    return outputs, hT
